# baseline (speedup 1.0000x reference)
import functools

import jax
import jax.numpy as jnp
from jax import lax
from jax.experimental import pallas as pl
from jax.experimental.pallas import tpu as pltpu
from jax.experimental.pallas import tpu_sc as plsc

V_MIN = -10.0
V_MAX = 10.0
NUM_ATOMS = 51
ATOM_DELTA = (V_MAX - V_MIN) / (NUM_ATOMS - 1)
INV_DELTA = 1.0 / ATOM_DELTA
BS = 16384
TOP_BIN = NUM_ATOMS - 1

NUM_CORES = 2
NUM_SUBCORES = 16
NUM_WORKERS = NUM_CORES * NUM_SUBCORES
COLS_PER_W = BS // NUM_WORKERS
GROUPS = COLS_PER_W // 16


def _sc_project(rw_hbm, probs_t_hbm, out_t_hbm, rws_v, probs_v, out_v, sem):
    wid = lax.axis_index("s") * NUM_CORES + lax.axis_index("c")
    base = wid * COLS_PER_W
    rw_cp = pltpu.async_copy(rw_hbm.at[pl.ds(base, COLS_PER_W)], rws_v, sem)
    p_cp = pltpu.async_copy(
        probs_t_hbm.at[:, pl.ds(base, COLS_PER_W)], probs_v, sem)

    zeros16 = jnp.zeros((16,), jnp.float32)
    lane = lax.iota(jnp.int32, 16)

    @plsc.parallel_loop(0, NUM_ATOMS, step=1, unroll=2)
    def zero_body(j):
        for g in range(GROUPS):
            out_v[j, pl.ds(g * 16, 16)] = zeros16

    rw_cp.wait()
    p_cp.wait()

    @plsc.parallel_loop(0, GROUPS, step=1, unroll=2)
    def group_body(g):
        col0 = g * 16
        cvec = lane + col0
        s16 = jnp.clip(rws_v[pl.ds(col0, 16)] * jnp.float32(INV_DELTA),
                       -52.0, 52.0)
        t = s16.astype(jnp.int32)
        m16 = t - (t.astype(jnp.float32) > s16).astype(jnp.int32)
        f16 = s16 - m16.astype(jnp.float32)
        omf = 1.0 - f16
        p_prev = probs_v[0, pl.ds(col0, 16)]
        plsc.addupdate_scatter(
            out_v, [jnp.clip(m16, 0, TOP_BIN), cvec], omf * p_prev)
        for k in range(1, NUM_ATOMS):
            t = jnp.clip(m16 + k, 0, TOP_BIN)
            p = probs_v[k, pl.ds(col0, 16)]
            plsc.addupdate_scatter(out_v, [t, cvec], f16 * p_prev + omf * p)
            p_prev = p
        plsc.addupdate_scatter(
            out_v, [jnp.clip(m16 + NUM_ATOMS, 0, TOP_BIN), cvec],
            f16 * p_prev)

    pltpu.sync_copy(out_v, out_t_hbm.at[:, pl.ds(base, COLS_PER_W)])


@functools.partial(
    pl.kernel,
    out_type=jax.ShapeDtypeStruct((NUM_ATOMS, BS), jnp.float32),
    mesh=plsc.VectorSubcoreMesh(core_axis_name="c", subcore_axis_name="s"),
    compiler_params=pltpu.CompilerParams(
        needs_layout_passes=False, use_tc_tiling_on_sc=True),
    scratch_types=[
        pltpu.VMEM((COLS_PER_W,), jnp.float32),
        pltpu.VMEM((NUM_ATOMS, COLS_PER_W), jnp.float32),
        pltpu.VMEM((NUM_ATOMS, COLS_PER_W), jnp.float32),
        pltpu.SemaphoreType.DMA,
    ],
)
def _projection_kernel(rw_hbm, probs_t_hbm, out_t_hbm,
                       rws_v, probs_v, out_v, sem):
    _sc_project(rw_hbm, probs_t_hbm, out_t_hbm, rws_v, probs_v, out_v, sem)


def kernel(reward, probs):
    return _projection_kernel(reward, probs.T).T
